# baseline (speedup 1.0000x reference)
import functools

import jax
import jax.numpy as jnp
from jax import lax
from jax.experimental import pallas as pl
from jax.experimental.pallas import tpu as pltpu
from jax.experimental.pallas import tpu_sc as plsc

NUM_EMBEDDINGS = 1024
EMBEDDING_DIM = 64
COMMITMENT_COST = 0.25
BATCH = 16
TOKENS = 576

N = BATCH * TOKENS
T = 4608
NB = N // T

NW = 32
B_PER_W = N // NW
GCH = 96


def _vq_block(x_ref, w_ref, idx_ref, wn_out_ref, loss_ref, wn_ref):
    i = pl.program_id(0)
    x = x_ref[...]

    @pl.when(i == 0)
    def _():
        w = w_ref[...]
        wn = w / jnp.maximum(
            jnp.sqrt(jnp.sum(w * w, axis=1, keepdims=True)), 1e-12)
        wn_ref[...] = wn
        wn_out_ref[...] = jnp.concatenate([wn, jnp.zeros_like(wn)], axis=1)

    wn = wn_ref[...]
    ssq = jnp.sum(x * x, axis=1, keepdims=True)
    m = jnp.maximum(jnp.sqrt(ssq), 1e-12)
    xn = x / m

    scores = lax.dot_general(xn, wn, (((1,), (1,)), ((), ())),
                             preferred_element_type=jnp.float32)
    d = 1.0 - scores
    idx = jnp.argmin(d, axis=1).astype(jnp.int32)
    dmin = jnp.min(d, axis=1)

    idx_ref[0, 0, :] = idx

    part = jnp.sum(1.0 + ssq[:, 0] - 2.0 * m[:, 0] * (1.0 - dmin))

    @pl.when(i == 0)
    def _():
        loss_ref[0, 0] = 0.0

    loss_ref[0, 0] += part

    @pl.when(i == NB - 1)
    def _():
        loss_ref[0, 0] = loss_ref[0, 0] * (COMMITMENT_COST / (N * EMBEDDING_DIM))


_sc_mesh = plsc.VectorSubcoreMesh(core_axis_name="c", subcore_axis_name="s")


@functools.partial(
    pl.kernel,
    mesh=_sc_mesh,
    out_type=jax.ShapeDtypeStruct((N, 2 * EMBEDDING_DIM), jnp.float32),
    scratch_types=[
        pltpu.VMEM((B_PER_W,), jnp.int32),
        pltpu.VMEM((B_PER_W, 2 * EMBEDDING_DIM), jnp.float32),
        pltpu.SemaphoreType.DMA,
    ],
)
def _sc_gather(wn_hbm, idx_hbm, out_hbm, idx_v, rows_v, sem):
    wid = lax.axis_index("s") * 2 + lax.axis_index("c")
    base = wid * B_PER_W
    pltpu.sync_copy(idx_hbm.at[pl.ds(base, B_PER_W)], idx_v)
    copies = [
        pltpu.async_copy(
            wn_hbm.at[idx_v.at[pl.ds(j * GCH, GCH)]],
            rows_v.at[pl.ds(j * GCH, GCH)],
            sem,
        )
        for j in range(B_PER_W // GCH)
    ]
    for c in copies:
        c.wait()
    pltpu.sync_copy(rows_v, out_hbm.at[pl.ds(base, B_PER_W)])


def kernel(inputs, W):
    flat = inputs.reshape(N, EMBEDDING_DIM)
    idx, wn, loss = pl.pallas_call(
        _vq_block,
        grid=(NB,),
        in_specs=[
            pl.BlockSpec((T, EMBEDDING_DIM), lambda i: (i, 0)),
            pl.BlockSpec((NUM_EMBEDDINGS, EMBEDDING_DIM), lambda i: (0, 0)),
        ],
        out_specs=[
            pl.BlockSpec((1, 1, T), lambda i: (i, 0, 0)),
            pl.BlockSpec((NUM_EMBEDDINGS, 2 * EMBEDDING_DIM),
                         lambda i: (0, 0)),
            pl.BlockSpec(memory_space=pltpu.SMEM, block_shape=(1, 1),
                         index_map=lambda i: (0, 0)),
        ],
        out_shape=[
            jax.ShapeDtypeStruct((NB, 1, T), jnp.int32),
            jax.ShapeDtypeStruct((NUM_EMBEDDINGS, 2 * EMBEDDING_DIM),
                                 jnp.float32),
            jax.ShapeDtypeStruct((1, 1), jnp.float32),
        ],
        scratch_shapes=[
            pltpu.VMEM((NUM_EMBEDDINGS, EMBEDDING_DIM), jnp.float32),
        ],
    )(flat, W)
    idx_flat = idx.reshape(N)
    qst = _sc_gather(wn, idx_flat)[:, :EMBEDDING_DIM]
    return (qst.reshape(inputs.shape), loss[0, 0],
            idx_flat.reshape(BATCH, TOKENS))


# speedup vs baseline: 1.5016x; 1.5016x over previous
import jax
import jax.numpy as jnp
from jax import lax
from jax.experimental import pallas as pl
from jax.experimental.pallas import tpu as pltpu

NUM_EMBEDDINGS = 1024
EMBEDDING_DIM = 64
COMMITMENT_COST = 0.25
BATCH = 16
TOKENS = 576

N = BATCH * TOKENS
T = 3072
NB = N // T


def _vq_block(x_ref, w_ref, qst_ref, idx_ref, loss_ref, wn_ref):
    i = pl.program_id(0)
    x = x_ref[...]

    @pl.when(i == 0)
    def _():
        w = w_ref[...]
        wn_ref[...] = w / jnp.maximum(
            jnp.sqrt(jnp.sum(w * w, axis=1, keepdims=True)), 1e-12)

    wn = wn_ref[...]
    ssq = jnp.sum(x * x, axis=1, keepdims=True)
    m = jnp.maximum(jnp.sqrt(ssq), 1e-12)
    xn = x / m

    scores = lax.dot_general(xn, wn, (((1,), (1,)), ((), ())),
                             preferred_element_type=jnp.float32)
    d = 1.0 - scores
    idx = jnp.argmin(d, axis=1).astype(jnp.int32)
    dmin = jnp.min(d, axis=1)

    onehot = (lax.broadcasted_iota(jnp.int32, (T, NUM_EMBEDDINGS), 1)
              == idx[:, None]).astype(jnp.float32)
    q = lax.dot_general(onehot, wn, (((1,), (0,)), ((), ())),
                        preferred_element_type=jnp.float32)

    qst_ref[...] = x + (q - x)
    idx_ref[0, 0, :] = idx

    part = jnp.sum(1.0 + ssq[:, 0] - 2.0 * m[:, 0] * (1.0 - dmin))

    @pl.when(i == 0)
    def _():
        loss_ref[0, 0] = 0.0

    loss_ref[0, 0] += part

    @pl.when(i == NB - 1)
    def _():
        loss_ref[0, 0] = loss_ref[0, 0] * (COMMITMENT_COST / (N * EMBEDDING_DIM))


def kernel(inputs, W):
    flat = inputs.reshape(N, EMBEDDING_DIM)
    qst, idx, loss = pl.pallas_call(
        _vq_block,
        grid=(NB,),
        in_specs=[
            pl.BlockSpec((T, EMBEDDING_DIM), lambda i: (i, 0)),
            pl.BlockSpec((NUM_EMBEDDINGS, EMBEDDING_DIM), lambda i: (0, 0)),
        ],
        out_specs=[
            pl.BlockSpec((T, EMBEDDING_DIM), lambda i: (i, 0)),
            pl.BlockSpec((1, 1, T), lambda i: (i, 0, 0)),
            pl.BlockSpec(memory_space=pltpu.SMEM, block_shape=(1, 1),
                         index_map=lambda i: (0, 0)),
        ],
        out_shape=[
            jax.ShapeDtypeStruct((N, EMBEDDING_DIM), jnp.float32),
            jax.ShapeDtypeStruct((NB, 1, T), jnp.int32),
            jax.ShapeDtypeStruct((1, 1), jnp.float32),
        ],
        scratch_shapes=[
            pltpu.VMEM((NUM_EMBEDDINGS, EMBEDDING_DIM), jnp.float32),
        ],
    )(flat, W)
    return (qst.reshape(inputs.shape), loss[0, 0],
            idx.reshape(BATCH, TOKENS))


# speedup vs baseline: 1.5227x; 1.0140x over previous
import jax
import jax.numpy as jnp
from jax import lax
from jax.experimental import pallas as pl
from jax.experimental.pallas import tpu as pltpu

NUM_EMBEDDINGS = 1024
EMBEDDING_DIM = 64
COMMITMENT_COST = 0.25
BATCH = 16
TOKENS = 576

N = BATCH * TOKENS
T = 4608
NB = N // T


def _vq_block(x_ref, w_ref, qst_ref, idx_ref, loss_ref, wn_ref):
    i = pl.program_id(0)
    x = x_ref[...]

    @pl.when(i == 0)
    def _():
        w = w_ref[...]
        wn_ref[...] = w / jnp.maximum(
            jnp.sqrt(jnp.sum(w * w, axis=1, keepdims=True)), 1e-12)

    wn = wn_ref[...]
    ssq = jnp.sum(x * x, axis=1, keepdims=True)
    m = jnp.maximum(jnp.sqrt(ssq), 1e-12)
    xn = x / m

    scores = lax.dot_general(xn, wn, (((1,), (1,)), ((), ())),
                             preferred_element_type=jnp.float32)
    d = 1.0 - scores
    idx = jnp.argmin(d, axis=1).astype(jnp.int32)
    dmin = jnp.min(d, axis=1)

    onehot = (lax.broadcasted_iota(jnp.int32, (T, NUM_EMBEDDINGS), 1)
              == idx[:, None]).astype(jnp.float32)
    q = lax.dot_general(onehot, wn, (((1,), (0,)), ((), ())),
                        preferred_element_type=jnp.float32)

    qst_ref[...] = x + (q - x)
    idx_ref[0, 0, :] = idx

    part = jnp.sum(1.0 + ssq[:, 0] - 2.0 * m[:, 0] * (1.0 - dmin))

    @pl.when(i == 0)
    def _():
        loss_ref[0, 0] = 0.0

    loss_ref[0, 0] += part

    @pl.when(i == NB - 1)
    def _():
        loss_ref[0, 0] = loss_ref[0, 0] * (COMMITMENT_COST / (N * EMBEDDING_DIM))


def kernel(inputs, W):
    flat = inputs.reshape(N, EMBEDDING_DIM)
    qst, idx, loss = pl.pallas_call(
        _vq_block,
        grid=(NB,),
        in_specs=[
            pl.BlockSpec((T, EMBEDDING_DIM), lambda i: (i, 0)),
            pl.BlockSpec((NUM_EMBEDDINGS, EMBEDDING_DIM), lambda i: (0, 0)),
        ],
        out_specs=[
            pl.BlockSpec((T, EMBEDDING_DIM), lambda i: (i, 0)),
            pl.BlockSpec((1, 1, T), lambda i: (i, 0, 0)),
            pl.BlockSpec(memory_space=pltpu.SMEM, block_shape=(1, 1),
                         index_map=lambda i: (0, 0)),
        ],
        out_shape=[
            jax.ShapeDtypeStruct((N, EMBEDDING_DIM), jnp.float32),
            jax.ShapeDtypeStruct((NB, 1, T), jnp.int32),
            jax.ShapeDtypeStruct((1, 1), jnp.float32),
        ],
        scratch_shapes=[
            pltpu.VMEM((NUM_EMBEDDINGS, EMBEDDING_DIM), jnp.float32),
        ],
    )(flat, W)
    return (qst.reshape(inputs.shape), loss[0, 0],
            idx.reshape(BATCH, TOKENS))


# speedup vs baseline: 1.6178x; 1.0625x over previous
import jax
import jax.numpy as jnp
from jax import lax
from jax.experimental import pallas as pl
from jax.experimental.pallas import tpu as pltpu

NUM_EMBEDDINGS = 1024
EMBEDDING_DIM = 64
COMMITMENT_COST = 0.25
BATCH = 16
TOKENS = 576

N = BATCH * TOKENS
T = 4608
NB = N // T


def _vq_block(x_ref, w_ref, qst_ref, idx_ref, loss_ref, wn_ref):
    i = pl.program_id(0)
    x = x_ref[...]

    @pl.when(i == 0)
    def _():
        w = w_ref[...]
        wn_ref[...] = w / jnp.maximum(
            jnp.sqrt(jnp.sum(w * w, axis=1, keepdims=True)), 1e-12)

    wn = wn_ref[...]
    ssq = jnp.sum(x * x, axis=1, keepdims=True)
    m = jnp.maximum(jnp.sqrt(ssq), 1e-12)
    xn = x / m

    scores = lax.dot_general(xn, wn, (((1,), (1,)), ((), ())),
                             preferred_element_type=jnp.float32)
    d = 1.0 - scores
    idx = jnp.argmin(d, axis=1).astype(jnp.int32)
    dmin = jnp.min(d, axis=1)

    onehot = (lax.broadcasted_iota(jnp.int32, (T, NUM_EMBEDDINGS), 1)
              == idx[:, None]).astype(jnp.float32)
    q = lax.dot_general(onehot, wn, (((1,), (0,)), ((), ())),
                        preferred_element_type=jnp.float32)

    qst_ref[...] = x + (q - x)
    idx_ref[...] = idx.reshape(T // TOKENS, TOKENS)

    part = jnp.sum(1.0 + ssq[:, 0] - 2.0 * m[:, 0] * (1.0 - dmin))

    @pl.when(i == 0)
    def _():
        loss_ref[0, 0] = 0.0

    loss_ref[0, 0] += part

    @pl.when(i == NB - 1)
    def _():
        loss_ref[0, 0] = loss_ref[0, 0] * (COMMITMENT_COST / (N * EMBEDDING_DIM))


def kernel(inputs, W):
    flat = inputs.reshape(N, EMBEDDING_DIM)
    qst, idx, loss = pl.pallas_call(
        _vq_block,
        grid=(NB,),
        in_specs=[
            pl.BlockSpec((T, EMBEDDING_DIM), lambda i: (i, 0)),
            pl.BlockSpec((NUM_EMBEDDINGS, EMBEDDING_DIM), lambda i: (0, 0)),
        ],
        out_specs=[
            pl.BlockSpec((T, EMBEDDING_DIM), lambda i: (i, 0)),
            pl.BlockSpec((T // TOKENS, TOKENS),
                         lambda i: (i, 0)),
            pl.BlockSpec(memory_space=pltpu.SMEM, block_shape=(1, 1),
                         index_map=lambda i: (0, 0)),
        ],
        out_shape=[
            jax.ShapeDtypeStruct((N, EMBEDDING_DIM), jnp.float32),
            jax.ShapeDtypeStruct((BATCH, TOKENS), jnp.int32),
            jax.ShapeDtypeStruct((1, 1), jnp.float32),
        ],
        scratch_shapes=[
            pltpu.VMEM((NUM_EMBEDDINGS, EMBEDDING_DIM), jnp.float32),
        ],
    )(flat, W)
    return (qst.reshape(inputs.shape), loss[0, 0], idx)


# speedup vs baseline: 1.6225x; 1.0029x over previous
import jax
import jax.numpy as jnp
from jax import lax
from jax.experimental import pallas as pl
from jax.experimental.pallas import tpu as pltpu

NUM_EMBEDDINGS = 1024
EMBEDDING_DIM = 64
COMMITMENT_COST = 0.25
BATCH = 16
TOKENS = 576

N = BATCH * TOKENS
T = 4608
NB = N // T


def _vq_block(x_ref, w_ref, qst_ref, idx_ref, loss_ref, wn_ref):
    i = pl.program_id(0)
    x = x_ref[...]

    @pl.when(i == 0)
    def _():
        w = w_ref[...]
        wn_ref[...] = w / jnp.maximum(
            jnp.sqrt(jnp.sum(w * w, axis=1, keepdims=True)), 1e-12)

    wn = wn_ref[...]
    ssq = jnp.sum(x * x, axis=1, keepdims=True)
    m = jnp.maximum(jnp.sqrt(ssq), 1e-12)
    xn = x / m

    scores = lax.dot_general(xn, wn, (((1,), (1,)), ((), ())),
                             preferred_element_type=jnp.float32)
    d = 1.0 - scores
    idx = jnp.argmin(d, axis=1).astype(jnp.int32)
    dmin = jnp.min(d, axis=1)

    onehot = (lax.broadcasted_iota(jnp.int32, (T, NUM_EMBEDDINGS), 1)
              == idx[:, None]).astype(jnp.float32)
    q = lax.dot_general(onehot, wn, (((1,), (0,)), ((), ())),
                        preferred_element_type=jnp.float32)

    qst_ref[...] = (x + (q - x)).reshape(T // TOKENS, TOKENS, EMBEDDING_DIM)
    idx_ref[...] = idx.reshape(T // TOKENS, TOKENS)

    part = jnp.sum(1.0 + ssq[:, 0] - 2.0 * m[:, 0] * (1.0 - dmin))

    @pl.when(i == 0)
    def _():
        loss_ref[0, 0] = 0.0

    loss_ref[0, 0] += part

    @pl.when(i == NB - 1)
    def _():
        loss_ref[0, 0] = loss_ref[0, 0] * (COMMITMENT_COST / (N * EMBEDDING_DIM))


def kernel(inputs, W):
    flat = inputs.reshape(N, EMBEDDING_DIM)
    qst, idx, loss = pl.pallas_call(
        _vq_block,
        grid=(NB,),
        in_specs=[
            pl.BlockSpec((T, EMBEDDING_DIM), lambda i: (i, 0)),
            pl.BlockSpec((NUM_EMBEDDINGS, EMBEDDING_DIM), lambda i: (0, 0)),
        ],
        out_specs=[
            pl.BlockSpec((T // TOKENS, TOKENS, EMBEDDING_DIM),
                         lambda i: (i, 0, 0)),
            pl.BlockSpec((T // TOKENS, TOKENS),
                         lambda i: (i, 0)),
            pl.BlockSpec(memory_space=pltpu.SMEM, block_shape=(1, 1),
                         index_map=lambda i: (0, 0)),
        ],
        out_shape=[
            jax.ShapeDtypeStruct((BATCH, TOKENS, EMBEDDING_DIM), jnp.float32),
            jax.ShapeDtypeStruct((BATCH, TOKENS), jnp.int32),
            jax.ShapeDtypeStruct((1, 1), jnp.float32),
        ],
        scratch_shapes=[
            pltpu.VMEM((NUM_EMBEDDINGS, EMBEDDING_DIM), jnp.float32),
        ],
    )(flat, W)
    return (qst, loss[0, 0], idx)
